# Initial kernel scaffold; baseline (speedup 1.0000x reference)
import jax
import jax.numpy as jnp
from jax.experimental import pallas as pl


def kernel(x, segmap_attentions, W, b):
    raise NotImplementedError("write your pallas kernel here")


# speedup vs baseline: 2.5164x; 2.5164x over previous
import functools

import jax
import jax.numpy as jnp
from jax.experimental import pallas as pl
from jax.experimental.pallas import tpu as pltpu

_B, _C, _H, _W = 4, 192, 384, 384
_J, _MH, _MW = 8, 96, 96
_FH, _FW = _H // _MH, _W // _MW
_HLT = 8


def _sums_body(x_ref, seg_ref, out_ref):
    t = pl.program_id(1)
    x2 = x_ref[0].reshape(_C, _FH * _HLT * _W)
    seg = seg_ref[0]
    mb = jnp.broadcast_to(
        seg[:, None, :, None, :], (_HLT, _FH, _MW, _FW, _J)
    ).reshape(_FH * _HLT * _W, _J)
    part = jax.lax.dot_general(
        x2, mb, (((1,), (0,)), ((), ())), preferred_element_type=jnp.float32
    )

    @pl.when(t == 0)
    def _():
        out_ref[...] = jnp.zeros_like(out_ref)

    out_ref[0] += part


def _finish_body(sums_ref, seg_ref, wt_ref, b_ref, out_ref):
    seg = seg_ref[...]
    area = jnp.sum(jnp.where(seg != 0, 1.0, 0.0), axis=(2, 3)) * (_FH * _FW)
    for j in range(_J):
        s = sums_ref[:, j, :]
        a = area[:, j]
        feat = s / jnp.maximum(a, 1.0)[:, None]
        h = jnp.maximum(feat, 0.0)
        o = (
            jax.lax.dot_general(
                h, wt_ref[j], (((1,), (0,)), ((), ())),
                preferred_element_type=jnp.float32,
            )
            + b_ref[j][None, :]
        )
        o = jnp.where((a > 0)[:, None], o, 0.0)
        out_ref[:, j, :] = o


@jax.jit
def kernel(x, segmap_attentions, W, b):
    seg_t = jnp.transpose(segmap_attentions, (0, 2, 3, 1))
    sums_cj = pl.pallas_call(
        _sums_body,
        grid=(_B, _MH // _HLT),
        in_specs=[
            pl.BlockSpec((1, _C, _FH * _HLT, _W), lambda b_, t: (b_, 0, t, 0)),
            pl.BlockSpec((1, _HLT, _MW, _J), lambda b_, t: (b_, t, 0, 0)),
        ],
        out_specs=pl.BlockSpec((1, _C, _J), lambda b_, t: (b_, 0, 0)),
        out_shape=jax.ShapeDtypeStruct((_B, _C, _J), jnp.float32),
        compiler_params=pltpu.CompilerParams(
            dimension_semantics=("parallel", "arbitrary"),
        ),
    )(x, seg_t)

    sums_jc = jnp.transpose(sums_cj, (0, 2, 1))
    wt = jnp.transpose(W, (0, 2, 1))
    out = pl.pallas_call(
        _finish_body,
        out_shape=jax.ShapeDtypeStruct((_B, _J, _C), jnp.float32),
    )(sums_jc, segmap_attentions, wt, b)
    return out


# speedup vs baseline: 2.6462x; 1.0516x over previous
import functools

import jax
import jax.numpy as jnp
from jax.experimental import pallas as pl
from jax.experimental.pallas import tpu as pltpu

_B, _C, _H, _W = 4, 192, 384, 384
_J, _MH, _MW = 8, 96, 96
_FH, _FW = _H // _MH, _W // _MW
_HLT = 16


def _sums_body(x_ref, seg_ref, out_ref):
    t = pl.program_id(1)
    x2 = x_ref[0].reshape(_C, _FH * _HLT * _W)
    seg = seg_ref[0]
    mb = jnp.broadcast_to(
        seg[:, None, :, None, :], (_HLT, _FH, _MW, _FW, _J)
    ).reshape(_FH * _HLT * _W, _J)
    part = jax.lax.dot_general(
        x2, mb, (((1,), (0,)), ((), ())), preferred_element_type=jnp.float32
    )

    @pl.when(t == 0)
    def _():
        out_ref[...] = jnp.zeros_like(out_ref)

    out_ref[0] += part


def _finish_body(sums_ref, seg_ref, wt_ref, b_ref, out_ref):
    seg = seg_ref[...]
    area = jnp.sum(jnp.where(seg != 0, 1.0, 0.0), axis=(2, 3)) * (_FH * _FW)
    for j in range(_J):
        s = sums_ref[:, j, :]
        a = area[:, j]
        feat = s / jnp.maximum(a, 1.0)[:, None]
        h = jnp.maximum(feat, 0.0)
        o = (
            jax.lax.dot_general(
                h, wt_ref[j], (((1,), (0,)), ((), ())),
                preferred_element_type=jnp.float32,
            )
            + b_ref[j][None, :]
        )
        o = jnp.where((a > 0)[:, None], o, 0.0)
        out_ref[:, j, :] = o


@jax.jit
def kernel(x, segmap_attentions, W, b):
    seg_t = jnp.transpose(segmap_attentions, (0, 2, 3, 1))
    sums_cj = pl.pallas_call(
        _sums_body,
        grid=(_B, _MH // _HLT),
        in_specs=[
            pl.BlockSpec((1, _C, _FH * _HLT, _W), lambda b_, t: (b_, 0, t, 0)),
            pl.BlockSpec((1, _HLT, _MW, _J), lambda b_, t: (b_, t, 0, 0)),
        ],
        out_specs=pl.BlockSpec((1, _C, _J), lambda b_, t: (b_, 0, 0)),
        out_shape=jax.ShapeDtypeStruct((_B, _C, _J), jnp.float32),
        compiler_params=pltpu.CompilerParams(
            dimension_semantics=("parallel", "arbitrary"),
        ),
    )(x, seg_t)

    sums_jc = jnp.transpose(sums_cj, (0, 2, 1))
    wt = jnp.transpose(W, (0, 2, 1))
    out = pl.pallas_call(
        _finish_body,
        out_shape=jax.ShapeDtypeStruct((_B, _J, _C), jnp.float32),
    )(sums_jc, segmap_attentions, wt, b)
    return out


# speedup vs baseline: 2.7377x; 1.0346x over previous
import functools

import jax
import jax.numpy as jnp
from jax.experimental import pallas as pl
from jax.experimental.pallas import tpu as pltpu

_B, _C, _H, _W = 4, 192, 384, 384
_J, _MH, _MW = 8, 96, 96
_FH, _FW = _H // _MH, _W // _MW
_HLT = 24


def _sums_body(x_ref, seg_ref, out_ref):
    t = pl.program_id(1)
    x2 = x_ref[0].reshape(_C, _FH * _HLT * _W)
    seg = seg_ref[0]
    mb = jnp.broadcast_to(
        seg[:, None, :, None, :], (_HLT, _FH, _MW, _FW, _J)
    ).reshape(_FH * _HLT * _W, _J)
    part = jax.lax.dot_general(
        x2, mb, (((1,), (0,)), ((), ())), preferred_element_type=jnp.float32
    )

    @pl.when(t == 0)
    def _():
        out_ref[...] = jnp.zeros_like(out_ref)

    out_ref[0] += part


def _finish_body(sums_ref, seg_ref, wt_ref, b_ref, out_ref):
    seg = seg_ref[...]
    area = jnp.sum(jnp.where(seg != 0, 1.0, 0.0), axis=(2, 3)) * (_FH * _FW)
    for j in range(_J):
        s = sums_ref[:, j, :]
        a = area[:, j]
        feat = s / jnp.maximum(a, 1.0)[:, None]
        h = jnp.maximum(feat, 0.0)
        o = (
            jax.lax.dot_general(
                h, wt_ref[j], (((1,), (0,)), ((), ())),
                preferred_element_type=jnp.float32,
            )
            + b_ref[j][None, :]
        )
        o = jnp.where((a > 0)[:, None], o, 0.0)
        out_ref[:, j, :] = o


@jax.jit
def kernel(x, segmap_attentions, W, b):
    seg_t = jnp.transpose(segmap_attentions, (0, 2, 3, 1))
    sums_cj = pl.pallas_call(
        _sums_body,
        grid=(_B, _MH // _HLT),
        in_specs=[
            pl.BlockSpec((1, _C, _FH * _HLT, _W), lambda b_, t: (b_, 0, t, 0)),
            pl.BlockSpec((1, _HLT, _MW, _J), lambda b_, t: (b_, t, 0, 0)),
        ],
        out_specs=pl.BlockSpec((1, _C, _J), lambda b_, t: (b_, 0, 0)),
        out_shape=jax.ShapeDtypeStruct((_B, _C, _J), jnp.float32),
        compiler_params=pltpu.CompilerParams(
            dimension_semantics=("parallel", "arbitrary"),
        ),
    )(x, seg_t)

    sums_jc = jnp.transpose(sums_cj, (0, 2, 1))
    wt = jnp.transpose(W, (0, 2, 1))
    out = pl.pallas_call(
        _finish_body,
        out_shape=jax.ShapeDtypeStruct((_B, _J, _C), jnp.float32),
    )(sums_jc, segmap_attentions, wt, b)
    return out
